# baseline (speedup 1.0000x reference)
import functools

import jax
import jax.numpy as jnp
import numpy as np
from jax import lax
from jax.experimental import pallas as pl
from jax.experimental.pallas import tpu as pltpu
from jax.experimental.pallas import tpu_sc as plsc

DEPTH = 10
N = 2047
NP = 2048
EMB = 128
ENC = 128
CHUNK = 128
NC = 2
NS = 16
NW = NC * NS
NBUF = 6

_PERM = np.zeros(NP, dtype=np.int64)
_PERM[1] = 0
for _l in range(1, DEPTH + 1):
    _k = 1 << (_l - 1)
    _par = _PERM[_k:2 * _k]
    _PERM[2 * _k:3 * _k] = 2 * _par + 1
    _PERM[3 * _k:4 * _k] = 2 * _par + 2
_INV = np.zeros(NP, dtype=np.int64)
_INV[_PERM[1:]] = np.arange(1, NP)
_INV[N] = 0

_B = 16
_r = np.arange(_B * NP)
_OUTPOS = ((_r // NP) * NP + _INV[_r % NP]).astype(np.int32).reshape(-1, CHUNK)


def _sc_gather_body(tok_hbm, pos_hbm, emb_hbm, out_hbm, idx_v, pos_v, bufs,
                    isem0, isem1, *sems):
    gs = sems[:NBUF]
    ss = sems[NBUF:]
    wid = lax.axis_index("s") * NC + lax.axis_index("c")
    n_chunks = (tok_hbm.shape[0] * tok_hbm.shape[1]) // (NW * CHUNK)
    tree = wid // 2
    col0 = (wid % 2) * (n_chunks * CHUNK)
    c0 = pltpu.async_copy(tok_hbm.at[tree, pl.ds(col0, n_chunks * CHUNK)],
                          idx_v, isem0)
    c1 = pltpu.async_copy(pos_hbm.at[pl.ds(wid * n_chunks, n_chunks)], pos_v,
                          isem1)
    c0.wait()
    c1.wait()
    h = [None] * n_chunks
    o = [None] * n_chunks
    prime = min(NBUF - 1, n_chunks)
    for j in range(prime):
        h[j] = pltpu.async_copy(emb_hbm.at[idx_v.at[pl.ds(j * CHUNK, CHUNK)]],
                                bufs.at[j], gs[j])
    for j in range(n_chunks):
        h[j].wait()
        o[j] = pltpu.async_copy(bufs.at[j % NBUF], out_hbm.at[pos_v.at[j]],
                                ss[j % NBUF])
        nj = j + prime
        if nj < n_chunks:
            if nj - NBUF >= 0:
                o[nj - NBUF].wait()
            h[nj] = pltpu.async_copy(
                emb_hbm.at[idx_v.at[pl.ds(nj * CHUNK, CHUNK)]],
                bufs.at[nj % NBUF], gs[nj % NBUF])
    for j in range(max(0, n_chunks - NBUF), n_chunks):
        o[j].wait()


def _sc_gather(tok_pad, pos_grid, emb):
    rows = tok_pad.shape[0] * tok_pad.shape[1]
    n_chunks = rows // (NW * CHUNK)
    kern = functools.partial(
        pl.kernel,
        out_type=jax.ShapeDtypeStruct((rows, EMB), jnp.float32),
        mesh=plsc.VectorSubcoreMesh(core_axis_name="c", subcore_axis_name="s"),
        scratch_types=[
            pltpu.VMEM((n_chunks * CHUNK,), jnp.int32),
            pltpu.VMEM((n_chunks, CHUNK), jnp.int32),
            pltpu.VMEM((NBUF, CHUNK, EMB), jnp.float32),
        ] + [pltpu.SemaphoreType.DMA] * (2 + 2 * NBUF),
    )(_sc_gather_body)
    return kern(tok_pad, pos_grid, emb)


TPB = 8


def _tc_body(g_ref, w_ref, b_ref, o_ref):
    w = w_ref[...].astype(jnp.bfloat16)
    bb = b_ref[...]
    for t in range(TPB):
        m = jnp.full((8, ENC), -jnp.inf, jnp.float32)
        carry = None
        for l in range(DEPTH, -1, -1):
            n = 1 << l
            g = g_ref[t, pl.ds(n, n), :]
            hb = lax.dot_general(g.astype(jnp.bfloat16), w,
                                 (((1,), (0,)), ((), ())),
                                 preferred_element_type=jnp.float32) + bb
            if carry is not None:
                hb = hb + carry
            if n >= 8:
                m = jnp.maximum(m, jnp.max(hb.reshape(n // 8, 8, ENC), axis=0))
            else:
                m = jnp.maximum(m, jnp.max(hb, axis=0, keepdims=True))
            if l > 0:
                k = n // 2
                carry = hb[:k] + hb[k:]
        o_ref[t] = jnp.max(m, axis=0, keepdims=True)


def _tc_encode(g3, W, b2):
    B = g3.shape[0]
    return pl.pallas_call(
        _tc_body,
        grid=(B // TPB,),
        in_specs=[
            pl.BlockSpec((TPB, NP, EMB), lambda i: (i, 0, 0)),
            pl.BlockSpec((EMB, ENC), lambda i: (0, 0)),
            pl.BlockSpec((1, ENC), lambda i: (0, 0)),
        ],
        out_specs=pl.BlockSpec((TPB, 1, ENC), lambda i: (i, 0, 0)),
        out_shape=jax.ShapeDtypeStruct((B, 1, ENC), jnp.float32),
    )(g3, W, b2)


def kernel(tokens, emb, W, b, bs):
    B, n = tokens.shape
    tok_pad = jnp.concatenate(
        [tokens, jnp.zeros((B, NP - n), jnp.int32)], axis=1)
    pos_grid = jnp.asarray(_OUTPOS)
    gathered = _sc_gather(tok_pad, pos_grid, emb)
    g3 = gathered.reshape(B, NP, EMB)
    return _tc_encode(g3, W, b.reshape(1, ENC)).reshape(B, ENC)


# speedup vs baseline: 1.0007x; 1.0007x over previous
import functools

import jax
import jax.numpy as jnp
import numpy as np
from jax import lax
from jax.experimental import pallas as pl
from jax.experimental.pallas import tpu as pltpu
from jax.experimental.pallas import tpu_sc as plsc

DEPTH = 10
N = 2047
NP = 2048
EMB = 128
ENC = 128
CHUNK = 128
NC = 2
NS = 16
NW = NC * NS
NBUF = 6

_PERM = np.zeros(NP, dtype=np.int64)
_PERM[1] = 0
for _l in range(1, DEPTH + 1):
    _k = 1 << (_l - 1)
    _par = _PERM[_k:2 * _k]
    _PERM[2 * _k:3 * _k] = 2 * _par + 1
    _PERM[3 * _k:4 * _k] = 2 * _par + 2
_INV = np.zeros(NP, dtype=np.int64)
_INV[_PERM[1:]] = np.arange(1, NP)
_INV[N] = 0

_B = 16
_r = np.arange(_B * NP)
_OUTPOS = ((_r // NP) * NP + _INV[_r % NP]).astype(np.int32).reshape(-1, CHUNK)


def _sc_gather_body(tok_hbm, pos_hbm, emb_hbm, out_hbm, idx_v, pos_v, bufs,
                    isem0, isem1, *sems):
    gs = sems[:NBUF]
    ss = sems[NBUF:]
    wid = lax.axis_index("s") * NC + lax.axis_index("c")
    n_chunks = (tok_hbm.shape[0] * tok_hbm.shape[1]) // (NW * CHUNK)
    tree = wid // 2
    col0 = (wid % 2) * (n_chunks * CHUNK)
    c0 = pltpu.async_copy(tok_hbm.at[tree, pl.ds(col0, n_chunks * CHUNK)],
                          idx_v, isem0)
    c1 = pltpu.async_copy(pos_hbm.at[pl.ds(wid * n_chunks, n_chunks)], pos_v,
                          isem1)
    c0.wait()
    c1.wait()
    h = [None] * n_chunks
    o = [None] * n_chunks
    prime = min(NBUF - 1, n_chunks)
    for j in range(prime):
        h[j] = pltpu.async_copy(emb_hbm.at[idx_v.at[pl.ds(j * CHUNK, CHUNK)]],
                                bufs.at[j], gs[j])
    for j in range(n_chunks):
        h[j].wait()
        o[j] = pltpu.async_copy(bufs.at[j % NBUF], out_hbm.at[pos_v.at[j]],
                                ss[j % NBUF])
        nj = j + prime
        if nj < n_chunks:
            if nj - NBUF >= 0:
                o[nj - NBUF].wait()
            h[nj] = pltpu.async_copy(
                emb_hbm.at[idx_v.at[pl.ds(nj * CHUNK, CHUNK)]],
                bufs.at[nj % NBUF], gs[nj % NBUF])
    for j in range(max(0, n_chunks - NBUF), n_chunks):
        o[j].wait()


def _sc_gather(tok_pad, pos_grid, emb):
    rows = tok_pad.shape[0] * tok_pad.shape[1]
    n_chunks = rows // (NW * CHUNK)
    kern = functools.partial(
        pl.kernel,
        out_type=jax.ShapeDtypeStruct((rows, EMB), jnp.float32),
        mesh=plsc.VectorSubcoreMesh(core_axis_name="c", subcore_axis_name="s"),
        scratch_types=[
            pltpu.VMEM((n_chunks * CHUNK,), jnp.int32),
            pltpu.VMEM((n_chunks, CHUNK), jnp.int32),
            pltpu.VMEM((NBUF, CHUNK, EMB), jnp.float32),
        ] + [pltpu.SemaphoreType.DMA] * (2 + 2 * NBUF),
    )(_sc_gather_body)
    return kern(tok_pad, pos_grid, emb)


TPB = 4


def _tc_body(g_ref, w_ref, b_ref, o_ref):
    w = w_ref[...].astype(jnp.bfloat16)
    bb = b_ref[...]
    for t in range(TPB):
        m = jnp.full((8, ENC), -jnp.inf, jnp.float32)
        carry = None
        for l in range(DEPTH, -1, -1):
            n = 1 << l
            g = g_ref[t, pl.ds(n, n), :]
            hb = lax.dot_general(g.astype(jnp.bfloat16), w,
                                 (((1,), (0,)), ((), ())),
                                 preferred_element_type=jnp.float32) + bb
            if carry is not None:
                hb = hb + carry
            if n >= 8:
                m = jnp.maximum(m, jnp.max(hb.reshape(n // 8, 8, ENC), axis=0))
            else:
                m = jnp.maximum(m, jnp.max(hb, axis=0, keepdims=True))
            if l > 0:
                k = n // 2
                carry = hb[:k] + hb[k:]
        o_ref[t] = jnp.max(m, axis=0, keepdims=True)


def _tc_encode(g3, W, b2):
    B = g3.shape[0]
    return pl.pallas_call(
        _tc_body,
        grid=(B // TPB,),
        in_specs=[
            pl.BlockSpec((TPB, NP, EMB), lambda i: (i, 0, 0)),
            pl.BlockSpec((EMB, ENC), lambda i: (0, 0)),
            pl.BlockSpec((1, ENC), lambda i: (0, 0)),
        ],
        out_specs=pl.BlockSpec((TPB, 1, ENC), lambda i: (i, 0, 0)),
        out_shape=jax.ShapeDtypeStruct((B, 1, ENC), jnp.float32),
    )(g3, W, b2)


def kernel(tokens, emb, W, b, bs):
    B, n = tokens.shape
    tok_pad = jnp.concatenate(
        [tokens, jnp.zeros((B, NP - n), jnp.int32)], axis=1)
    pos_grid = jnp.asarray(_OUTPOS)
    gathered = _sc_gather(tok_pad, pos_grid, emb)
    g3 = gathered.reshape(B, NP, EMB)
    return _tc_encode(g3, W, b.reshape(1, ENC)).reshape(B, ENC)


# speedup vs baseline: 1.0402x; 1.0395x over previous
import functools

import jax
import jax.numpy as jnp
import numpy as np
from jax import lax
from jax.experimental import pallas as pl
from jax.experimental.pallas import tpu as pltpu
from jax.experimental.pallas import tpu_sc as plsc

DEPTH = 10
N = 2047
NP = 2048
EMB = 128
ENC = 128
CHUNK = 128
NC = 2
NS = 16
NW = NC * NS
NBUF = 6

_PERM = np.zeros(NP, dtype=np.int64)
_PERM[1] = 0
for _l in range(1, DEPTH + 1):
    _k = 1 << (_l - 1)
    _par = _PERM[_k:2 * _k]
    _PERM[2 * _k:3 * _k] = 2 * _par + 1
    _PERM[3 * _k:4 * _k] = 2 * _par + 2
_INV = np.zeros(NP, dtype=np.int64)
_INV[_PERM[1:]] = np.arange(1, NP)
_INV[N] = 0

_B = 16
_OUTPOS = np.zeros((_B * NP // CHUNK, CHUNK), dtype=np.int32)
for _w in range(NW):
    _R, _C = _w % 2, _w // 2
    for _rr in range(8):
        _tree = 8 * _R + _rr
        _node = 128 * _C + np.arange(CHUNK)
        _OUTPOS[_w * 8 + _rr] = _tree * NP + _INV[_node]


def _sc_gather_body(tok_hbm, pos_hbm, emb_hbm, out_hbm, idx_v, pos_v, bufs,
                    isem0, isem1, *sems):
    gs = sems[:NBUF]
    ss = sems[NBUF:]
    wid = lax.axis_index("s") * NC + lax.axis_index("c")
    n_chunks = (tok_hbm.shape[0] * NP) // (NW * CHUNK)
    rtile = pl.multiple_of((wid % 2) * 8, 8)
    ctile = pl.multiple_of((wid // 2) * CHUNK, CHUNK)
    c1 = pltpu.async_copy(pos_hbm.at[pl.ds(wid * n_chunks, n_chunks)], pos_v,
                          isem1)
    pltpu.sync_copy(tok_hbm.at[pl.ds(rtile, 8), pl.ds(ctile, CHUNK)], idx_v)
    @pl.when(wid // 2 == (NP // CHUNK) - 1)
    def _():
        lane = lax.iota(jnp.int32, 16)
        for r in range(8):
            v = idx_v[r, pl.ds(CHUNK - 16, 16)]
            idx_v[r, pl.ds(CHUNK - 16, 16)] = jnp.where(lane == 15, 0, v)

    c1.wait()
    h = [None] * n_chunks
    o = [None] * n_chunks
    prime = min(NBUF - 1, n_chunks)
    for j in range(prime):
        h[j] = pltpu.async_copy(emb_hbm.at[idx_v.at[j]], bufs.at[j], gs[j])
    for j in range(n_chunks):
        h[j].wait()
        o[j] = pltpu.async_copy(bufs.at[j % NBUF], out_hbm.at[pos_v.at[j]],
                                ss[j % NBUF])
        nj = j + prime
        if nj < n_chunks:
            if nj - NBUF >= 0:
                o[nj - NBUF].wait()
            h[nj] = pltpu.async_copy(emb_hbm.at[idx_v.at[nj]],
                                     bufs.at[nj % NBUF], gs[nj % NBUF])
    for j in range(max(0, n_chunks - NBUF), n_chunks):
        o[j].wait()


def _sc_gather(tok_pad, pos_grid, emb):
    rows = tok_pad.shape[0] * NP
    n_chunks = rows // (NW * CHUNK)
    kern = functools.partial(
        pl.kernel,
        out_type=jax.ShapeDtypeStruct((rows, EMB), jnp.float32),
        mesh=plsc.VectorSubcoreMesh(core_axis_name="c", subcore_axis_name="s"),
        scratch_types=[
            pltpu.VMEM((n_chunks, CHUNK), jnp.int32),
            pltpu.VMEM((n_chunks, CHUNK), jnp.int32),
            pltpu.VMEM((NBUF, CHUNK, EMB), jnp.float32),
        ] + [pltpu.SemaphoreType.DMA] * (2 + 2 * NBUF),
    )(_sc_gather_body)
    return kern(tok_pad, pos_grid, emb)


TPB = 4


def _tc_body(g_ref, w_ref, b_ref, o_ref):
    w = w_ref[...].astype(jnp.bfloat16)
    bb = b_ref[...]
    for t in range(TPB):
        m = jnp.full((8, ENC), -jnp.inf, jnp.float32)
        carry = None
        for l in range(DEPTH, -1, -1):
            n = 1 << l
            g = g_ref[t, pl.ds(n, n), :]
            hb = lax.dot_general(g.astype(jnp.bfloat16), w,
                                 (((1,), (0,)), ((), ())),
                                 preferred_element_type=jnp.float32) + bb
            if carry is not None:
                hb = hb + carry
            if n >= 8:
                m = jnp.maximum(m, jnp.max(hb.reshape(n // 8, 8, ENC), axis=0))
            else:
                m = jnp.maximum(m, jnp.max(hb, axis=0, keepdims=True))
            if l > 0:
                k = n // 2
                carry = hb[:k] + hb[k:]
        o_ref[t] = jnp.max(m, axis=0, keepdims=True)


def _tc_encode(g3, W, b2):
    B = g3.shape[0]
    return pl.pallas_call(
        _tc_body,
        grid=(B // TPB,),
        in_specs=[
            pl.BlockSpec((TPB, NP, EMB), lambda i: (i, 0, 0)),
            pl.BlockSpec((EMB, ENC), lambda i: (0, 0)),
            pl.BlockSpec((1, ENC), lambda i: (0, 0)),
        ],
        out_specs=pl.BlockSpec((TPB, 1, ENC), lambda i: (i, 0, 0)),
        out_shape=jax.ShapeDtypeStruct((B, 1, ENC), jnp.float32),
    )(g3, W, b2)


def kernel(tokens, emb, W, b, bs):
    B, n = tokens.shape
    pos_grid = jnp.asarray(_OUTPOS)
    gathered = _sc_gather(tokens, pos_grid, emb)
    g3 = gathered.reshape(B, NP, EMB)
    return _tc_encode(g3, W, b.reshape(1, ENC)).reshape(B, ENC)
